# baseline (speedup 1.0000x reference)
import numpy as np
import jax
import jax.numpy as jnp
from jax.experimental import pallas as pl
from jax.experimental.pallas import tpu as pltpu

_B = 4
_T = 24
_HH = 32
_WW = 32
_D = 64
_NE = 7
_FPAD = 16
_TOK = _T * _HH * _WW

_t = np.arange(_T)
_f = np.arange(1, _T // 2 + 1)
_ang = 2.0 * np.pi * _f[:, None] * _t[None, :] / _T
_Cp = np.zeros((_FPAD, _T), np.float32)
_Sp = np.zeros((_FPAD, _T), np.float32)
_Cp[: _T // 2] = (np.cos(_ang) / np.sqrt(_T)).astype(np.float32)
_Sp[: _T // 2] = (np.sin(_ang) / np.sqrt(_T)).astype(np.float32)
_CBIG = np.zeros((_B * _FPAD, _B * _T), np.float32)
_SBIG = np.zeros((_B * _FPAD, _B * _T), np.float32)
for _b in range(_B):
    _CBIG[_b * _FPAD:(_b + 1) * _FPAD, _b * _T:(_b + 1) * _T] = _Cp
    _SBIG[_b * _FPAD:(_b + 1) * _FPAD, _b * _T:(_b + 1) * _T] = _Sp


def _morton(x):
    n, hh, ww, c = x.shape
    x = x.reshape(n, 2, 2, 2, 2, 2, 2, 2, 2, 2, 2, c)
    x = x.transpose(0, 1, 6, 2, 7, 3, 8, 4, 9, 5, 10, 11)
    return x.reshape(n * hh * ww, c)


def _ln_gelu(h, g, beta):
    mu = jnp.mean(h, axis=1, keepdims=True)
    var = jnp.mean((h - mu) ** 2, axis=1, keepdims=True)
    hn = (h - mu) * jax.lax.rsqrt(var + 1e-5)
    return jax.nn.gelu(hn * g + beta)


_ROWS_A = (2048, 512, 128, 32)


def _gate_mega_body(p_ref, w1, w2, w3, w4, b1, b2, b3, b4,
                    g1, g2, g3, g4, t1, t2, t3, t4,
                    w5_ref, b5, g5, t5, fw_ref, fb_ref,
                    cb_ref, sb_ref, wg_ref, idx_ref, gts_ref,
                    h4_scr, h5_scr, amp_scr):
    i = pl.program_id(0)

    @pl.when(i < 12)
    def _tower():
        v = p_ref[...]
        for li, (w, b, g, t) in enumerate(
                ((w1, b1, g1, t1), (w2, b2, g2, t2),
                 (w3, b3, g3, t3), (w4, b4, g4, t4))):
            if li > 0:
                v = v.reshape(_ROWS_A[li], v.shape[1] * 4)
            h = jnp.dot(v, w[...], preferred_element_type=jnp.float32) + b[...]
            v = _ln_gelu(h, g[...], t[...])
        h4_scr[pl.ds(8 * i, 8), :] = v.reshape(8, 4096)

    for jj in range(8):
        @pl.when(i == 12 + jj)
        def _l5(jj=jj):
            vc = h4_scr[:, 512 * jj:512 * (jj + 1)]
            part = jnp.dot(vc, w5_ref[...], preferred_element_type=jnp.float32)
            if jj == 0:
                h5_scr[...] = part
            else:
                h5_scr[...] += part
            if jj == 7:
                h5_scr[...] = _ln_gelu(h5_scr[...] + b5[...], g5[...], t5[...])

    for jj in range(8):
        @pl.when(i == 20 + jj)
        def _fuse(jj=jj):
            fc = jax.lax.dot_general(
                h5_scr[...], fw_ref[...], (((1,), (1,)), ((), ())),
                preferred_element_type=jnp.float32)
            fc = fc + fb_ref[:, 256 * jj:256 * (jj + 1)]
            re = jnp.dot(cb_ref[...], fc, preferred_element_type=jnp.float32)
            im = jnp.dot(sb_ref[...], fc, preferred_element_type=jnp.float32)
            mag = jnp.sqrt(re * re + im * im)
            part = jnp.sum(mag.reshape(_B * _FPAD, 2, 128), axis=1)
            if jj == 0:
                amp_scr[...] = part
            else:
                amp_scr[...] += part
            if jj == 7:
                amp = jnp.sum(amp_scr[...], axis=1, keepdims=True) / 2048.0
                ii = jax.lax.broadcasted_iota(jnp.int32, (1, _NE), 1)
                for b in range(_B):
                    a_b = amp[_FPAD * b:_FPAD * b + _T // 2]
                    lg = jnp.sum(a_b * wg_ref[...], axis=0, keepdims=True)
                    m1 = jnp.max(lg)
                    i1 = jnp.min(jnp.where(lg == m1, ii, _NE))
                    lg2 = jnp.where(ii == i1, jnp.float32(-1e30), lg)
                    m2 = jnp.max(lg2)
                    i2 = jnp.min(jnp.where(lg2 == m2, ii, _NE))
                    d = jnp.exp(m2 - m1)
                    idx_ref[2 * b] = i1
                    idx_ref[2 * b + 1] = i2
                    gts_ref[2 * b] = 1.0 / (1.0 + d)
                    gts_ref[2 * b + 1] = d / (1.0 + d)


def _c0(shape):
    n = len(shape)
    return pl.BlockSpec(shape, lambda i, _n=n: (0,) * _n)


def _expert_body(idx_ref, gts_ref, x_ref, w0_ref, w1_ref, b0_ref, b1_ref, o_ref):
    b = pl.program_id(0)
    xb = x_ref[0]
    w = jnp.concatenate([w0_ref[0], w1_ref[0]], axis=1)
    a = jnp.dot(xb.astype(jnp.bfloat16), w.astype(jnp.bfloat16),
                preferred_element_type=jnp.float32)
    g0 = gts_ref[2 * b]
    g1 = gts_ref[2 * b + 1]
    a0 = a[:, :_D] + (b0_ref[0] + jnp.log(g0))
    a1 = a[:, _D:] + (b1_ref[0] + jnp.log(g1))
    o_ref[0] = jnp.logaddexp(a0, a1)


def _full(shape):
    return pl.BlockSpec(shape, lambda i: (0,) * len(shape))


def kernel(x, params):
    h0 = _morton(x.reshape(_B * _T, _HH, _WW, _D))
    p1 = h0.reshape(_B * _T * _HH * _WW // 4, 4 * _D)

    wms, b2s, g2s, t2s = [], [], [], []
    for i in range(5):
        cw = params["conv_w"][i]
        wms.append(cw.transpose(2, 3, 1, 0).reshape(-1, cw.shape[0]))
        b2s.append(params["conv_b"][i].reshape(1, -1))
        g2s.append(params["ln_g"][i].reshape(1, -1))
        t2s.append(params["ln_b"][i].reshape(1, -1))

    in_specs_a = [pl.BlockSpec((2048, 256), lambda i: (jnp.minimum(i, 11), 0))]
    for arrs in (wms[:4], b2s[:4], g2s[:4], t2s[:4]):
        for a in arrs:
            in_specs_a.append(_c0(a.shape))
    in_specs_a += [
        pl.BlockSpec((512, 2048), lambda i: (jnp.clip(i - 12, 0, 7), 0)),
        _c0((1, 2048)), _c0((1, 2048)), _c0((1, 2048)),
        pl.BlockSpec((256, 2048), lambda i: (jnp.clip(i - 20, 0, 7), 0)),
        _c0((1, 2048)),
        _c0((_B * _FPAD, _B * _T)), _c0((_B * _FPAD, _B * _T)),
        _c0((_T // 2, _NE)),
    ]
    idx8, gts8 = pl.pallas_call(
        _gate_mega_body,
        grid=(28,),
        in_specs=in_specs_a,
        out_specs=(pl.BlockSpec(memory_space=pltpu.SMEM),
                   pl.BlockSpec(memory_space=pltpu.SMEM)),
        out_shape=(jax.ShapeDtypeStruct((2 * _B,), jnp.int32),
                   jax.ShapeDtypeStruct((2 * _B,), jnp.float32)),
        scratch_shapes=[
            pltpu.VMEM((_B * _T, 4096), jnp.float32),
            pltpu.VMEM((_B * _T, 2048), jnp.float32),
            pltpu.VMEM((_B * _FPAD, 128), jnp.float32),
        ],
    )(p1, *wms[:4], *b2s[:4], *g2s[:4], *t2s[:4],
      wms[4], b2s[4], g2s[4], t2s[4], params["fuse_w"],
      params["fuse_b"].reshape(1, -1), _CBIG, _SBIG, params["w_gate"])
    x3 = x.reshape(_B, _TOK, _D)
    eb3 = params["expert_b"].reshape(_NE, 1, _D)
    tt = 8192
    grid_spec = pltpu.PrefetchScalarGridSpec(
        num_scalar_prefetch=2,
        grid=(_B, _TOK // tt),
        in_specs=[
            pl.BlockSpec((1, tt, _D), lambda b, t, idx, gts: (b, t, 0)),
            pl.BlockSpec((1, _D, _D), lambda b, t, idx, gts: (idx[2 * b], 0, 0)),
            pl.BlockSpec((1, _D, _D), lambda b, t, idx, gts: (idx[2 * b + 1], 0, 0)),
            pl.BlockSpec((1, 1, _D), lambda b, t, idx, gts: (idx[2 * b], 0, 0)),
            pl.BlockSpec((1, 1, _D), lambda b, t, idx, gts: (idx[2 * b + 1], 0, 0)),
        ],
        out_specs=pl.BlockSpec((1, tt, _D), lambda b, t, idx, gts: (b, t, 0)),
    )
    out = pl.pallas_call(
        _expert_body,
        grid_spec=grid_spec,
        out_shape=jax.ShapeDtypeStruct((_B, _TOK, _D), jnp.float32),
    )(idx8, gts8, x3, params["expert_w"], params["expert_w"], eb3, eb3)
    return out.reshape(_B, _T, _HH, _WW, _D)


# speedup vs baseline: 1.0506x; 1.0506x over previous
import numpy as np
import jax
import jax.numpy as jnp
from jax.experimental import pallas as pl
from jax.experimental.pallas import tpu as pltpu

_B = 4
_T = 24
_HH = 32
_WW = 32
_D = 64
_NE = 7
_FPAD = 16
_TOK = _T * _HH * _WW

_t = np.arange(_T)
_f = np.arange(1, _T // 2 + 1)
_ang = 2.0 * np.pi * _f[:, None] * _t[None, :] / _T
_Cp = np.zeros((_FPAD, _T), np.float32)
_Sp = np.zeros((_FPAD, _T), np.float32)
_Cp[: _T // 2] = (np.cos(_ang) / np.sqrt(_T)).astype(np.float32)
_Sp[: _T // 2] = (np.sin(_ang) / np.sqrt(_T)).astype(np.float32)
_CBIG = np.zeros((_B * _FPAD, _B * _T), np.float32)
_SBIG = np.zeros((_B * _FPAD, _B * _T), np.float32)
for _b in range(_B):
    _CBIG[_b * _FPAD:(_b + 1) * _FPAD, _b * _T:(_b + 1) * _T] = _Cp
    _SBIG[_b * _FPAD:(_b + 1) * _FPAD, _b * _T:(_b + 1) * _T] = _Sp


def _morton(x):
    n, hh, ww, c = x.shape
    x = x.reshape(n, 2, 2, 2, 2, 2, 2, 2, 2, 2, 2, c)
    x = x.transpose(0, 1, 6, 2, 7, 3, 8, 4, 9, 5, 10, 11)
    return x.reshape(n * hh * ww, c)


def _ln_gelu(h, g, beta):
    mu = jnp.mean(h, axis=1, keepdims=True)
    var = jnp.mean((h - mu) ** 2, axis=1, keepdims=True)
    hn = (h - mu) * jax.lax.rsqrt(var + 1e-5)
    return jax.nn.gelu(hn * g + beta)


_ROWS_A = (4096, 1024, 256, 64)


def _tower_body(p_ref, w1, w2, w3, w4, b1, b2, b3, b4,
                g1, g2, g3, g4, t1, t2, t3, t4, o_ref):
    v = p_ref[...]
    for li, (w, b, g, t) in enumerate(
            ((w1, b1, g1, t1), (w2, b2, g2, t2),
             (w3, b3, g3, t3), (w4, b4, g4, t4))):
        if li > 0:
            v = v.reshape(_ROWS_A[li], v.shape[1] * 4)
        h = jnp.dot(v, w[...], preferred_element_type=jnp.float32) + b[...]
        v = _ln_gelu(h, g[...], t[...])
    o_ref[...] = v


def _head_body(p_ref, w5, b5, g5, t5, fw_ref, fb_ref,
               cb_ref, sb_ref, wg_ref, idx_ref, gts_ref):
    v = p_ref[...].reshape(_B * _T, 4096)
    h5 = _ln_gelu(jnp.dot(v, w5[...], preferred_element_type=jnp.float32)
                  + b5[...], g5[...], t5[...])
    fused = jax.lax.dot_general(
        h5, fw_ref[...], (((1,), (1,)), ((), ())),
        preferred_element_type=jnp.float32) + fb_ref[...]
    re = jnp.dot(cb_ref[...], fused, preferred_element_type=jnp.float32)
    im = jnp.dot(sb_ref[...], fused, preferred_element_type=jnp.float32)
    amp = jnp.mean(jnp.sqrt(re * re + im * im), axis=1, keepdims=True)
    ii = jax.lax.broadcasted_iota(jnp.int32, (1, _NE), 1)
    for b in range(_B):
        a_b = amp[_FPAD * b:_FPAD * b + _T // 2]
        lg = jnp.sum(a_b * wg_ref[...], axis=0, keepdims=True)
        m1 = jnp.max(lg)
        i1 = jnp.min(jnp.where(lg == m1, ii, _NE))
        lg2 = jnp.where(ii == i1, jnp.float32(-1e30), lg)
        m2 = jnp.max(lg2)
        i2 = jnp.min(jnp.where(lg2 == m2, ii, _NE))
        d = jnp.exp(m2 - m1)
        idx_ref[2 * b] = i1
        idx_ref[2 * b + 1] = i2
        gts_ref[2 * b] = 1.0 / (1.0 + d)
        gts_ref[2 * b + 1] = d / (1.0 + d)


def _expert_body(idx_ref, gts_ref, x_ref, w0_ref, w1_ref, b0_ref, b1_ref, o_ref):
    b = pl.program_id(0)
    xb = x_ref[0]
    w = jnp.concatenate([w0_ref[0], w1_ref[0]], axis=1)
    a = jnp.dot(xb.astype(jnp.bfloat16), w.astype(jnp.bfloat16),
                preferred_element_type=jnp.float32)
    g0 = gts_ref[2 * b]
    g1 = gts_ref[2 * b + 1]
    a0 = a[:, :_D] + (b0_ref[0] + jnp.log(g0))
    a1 = a[:, _D:] + (b1_ref[0] + jnp.log(g1))
    o_ref[0] = jnp.logaddexp(a0, a1)


def _full(shape):
    return pl.BlockSpec(shape, lambda i: (0,) * len(shape))


def kernel(x, params):
    h0 = _morton(x.reshape(_B * _T, _HH, _WW, _D))
    p1 = h0.reshape(_B * _T * _HH * _WW // 4, 4 * _D)

    wms, b2s, g2s, t2s = [], [], [], []
    for i in range(5):
        cw = params["conv_w"][i]
        wms.append(cw.transpose(2, 3, 1, 0).reshape(-1, cw.shape[0]))
        b2s.append(params["conv_b"][i].reshape(1, -1))
        g2s.append(params["ln_g"][i].reshape(1, -1))
        t2s.append(params["ln_b"][i].reshape(1, -1))

    in_specs_a = [pl.BlockSpec((4096, 256), lambda i: (i, 0))]
    for arrs in (wms[:4], b2s[:4], g2s[:4], t2s[:4]):
        for a in arrs:
            in_specs_a.append(_full(a.shape))
    h4 = pl.pallas_call(
        _tower_body,
        grid=(6,),
        in_specs=in_specs_a,
        out_specs=pl.BlockSpec((64, 1024), lambda i: (i, 0)),
        out_shape=jax.ShapeDtypeStruct((_B * _T * 4, 1024), jnp.float32),
    )(p1, *wms[:4], *b2s[:4], *g2s[:4], *t2s[:4])

    head_in = [h4, wms[4], b2s[4], g2s[4], t2s[4], params["fuse_w"],
               params["fuse_b"].reshape(1, -1), _CBIG, _SBIG, params["w_gate"]]
    idx8, gts8 = pl.pallas_call(
        _head_body,
        out_specs=(pl.BlockSpec(memory_space=pltpu.SMEM),
                   pl.BlockSpec(memory_space=pltpu.SMEM)),
        out_shape=(jax.ShapeDtypeStruct((2 * _B,), jnp.int32),
                   jax.ShapeDtypeStruct((2 * _B,), jnp.float32)),
    )(*head_in)
    x3 = x.reshape(_B, _TOK, _D)
    eb3 = params["expert_b"].reshape(_NE, 1, _D)
    tt = 12288
    grid_spec = pltpu.PrefetchScalarGridSpec(
        num_scalar_prefetch=2,
        grid=(_B, _TOK // tt),
        in_specs=[
            pl.BlockSpec((1, tt, _D), lambda b, t, idx, gts: (b, t, 0)),
            pl.BlockSpec((1, _D, _D), lambda b, t, idx, gts: (idx[2 * b], 0, 0)),
            pl.BlockSpec((1, _D, _D), lambda b, t, idx, gts: (idx[2 * b + 1], 0, 0)),
            pl.BlockSpec((1, 1, _D), lambda b, t, idx, gts: (idx[2 * b], 0, 0)),
            pl.BlockSpec((1, 1, _D), lambda b, t, idx, gts: (idx[2 * b + 1], 0, 0)),
        ],
        out_specs=pl.BlockSpec((1, tt, _D), lambda b, t, idx, gts: (b, t, 0)),
    )
    out = pl.pallas_call(
        _expert_body,
        grid_spec=grid_spec,
        out_shape=jax.ShapeDtypeStruct((_B, _TOK, _D), jnp.float32),
    )(idx8, gts8, x3, params["expert_w"], params["expert_w"], eb3, eb3)
    return out.reshape(_B, _T, _HH, _WW, _D)


# speedup vs baseline: 1.0652x; 1.0139x over previous
import numpy as np
import jax
import jax.numpy as jnp
from jax.experimental import pallas as pl
from jax.experimental.pallas import tpu as pltpu

_B = 4
_T = 24
_HH = 32
_WW = 32
_D = 64
_NE = 7
_FPAD = 16
_TOK = _T * _HH * _WW

_t = np.arange(_T)
_f = np.arange(1, _T // 2 + 1)
_ang = 2.0 * np.pi * _f[:, None] * _t[None, :] / _T
_Cp = np.zeros((_FPAD, _T), np.float32)
_Sp = np.zeros((_FPAD, _T), np.float32)
_Cp[: _T // 2] = (np.cos(_ang) / np.sqrt(_T)).astype(np.float32)
_Sp[: _T // 2] = (np.sin(_ang) / np.sqrt(_T)).astype(np.float32)
_CBIG = np.zeros((_B * _FPAD, _B * _T), np.float32)
_SBIG = np.zeros((_B * _FPAD, _B * _T), np.float32)
for _b in range(_B):
    _CBIG[_b * _FPAD:(_b + 1) * _FPAD, _b * _T:(_b + 1) * _T] = _Cp
    _SBIG[_b * _FPAD:(_b + 1) * _FPAD, _b * _T:(_b + 1) * _T] = _Sp


def _morton(x):
    n, hh, ww, c = x.shape
    x = x.reshape(n, 2, 2, 2, 2, 2, 2, 2, 2, 2, 2, c)
    x = x.transpose(0, 1, 6, 2, 7, 3, 8, 4, 9, 5, 10, 11)
    return x.reshape(n * hh * ww, c)


def _ln_gelu(h, g, beta):
    mu = jnp.mean(h, axis=1, keepdims=True)
    var = jnp.mean((h - mu) ** 2, axis=1, keepdims=True)
    hn = (h - mu) * jax.lax.rsqrt(var + 1e-5)
    return jax.nn.gelu(hn * g + beta)


_ROWS_A = (8192, 2048, 512, 128)


def _tower_body(p_ref, w1, w2, w3, w4, b1, b2, b3, b4,
                g1, g2, g3, g4, t1, t2, t3, t4, o_ref):
    v = p_ref[...]
    for li, (w, b, g, t) in enumerate(
            ((w1, b1, g1, t1), (w2, b2, g2, t2),
             (w3, b3, g3, t3), (w4, b4, g4, t4))):
        if li > 0:
            v = v.reshape(_ROWS_A[li], v.shape[1] * 4)
        h = jnp.dot(v, w[...], preferred_element_type=jnp.float32) + b[...]
        v = _ln_gelu(h, g[...], t[...])
    o_ref[...] = v


def _head_body(p_ref, w5, b5, g5, t5, fw_ref, fb_ref,
               cb_ref, sb_ref, wg_ref, idx_ref, gts_ref):
    v = p_ref[...].reshape(_B * _T, 4096)
    h5 = _ln_gelu(jnp.dot(v, w5[...], preferred_element_type=jnp.float32)
                  + b5[...], g5[...], t5[...])
    fused = jax.lax.dot_general(
        h5, fw_ref[...], (((1,), (1,)), ((), ())),
        preferred_element_type=jnp.float32) + fb_ref[...]
    re = jnp.dot(cb_ref[...], fused, preferred_element_type=jnp.float32)
    im = jnp.dot(sb_ref[...], fused, preferred_element_type=jnp.float32)
    amp = jnp.mean(jnp.sqrt(re * re + im * im), axis=1, keepdims=True)
    ii = jax.lax.broadcasted_iota(jnp.int32, (1, _NE), 1)
    for b in range(_B):
        a_b = amp[_FPAD * b:_FPAD * b + _T // 2]
        lg = jnp.sum(a_b * wg_ref[...], axis=0, keepdims=True)
        m1 = jnp.max(lg)
        i1 = jnp.min(jnp.where(lg == m1, ii, _NE))
        lg2 = jnp.where(ii == i1, jnp.float32(-1e30), lg)
        m2 = jnp.max(lg2)
        i2 = jnp.min(jnp.where(lg2 == m2, ii, _NE))
        d = jnp.exp(m2 - m1)
        idx_ref[2 * b] = i1
        idx_ref[2 * b + 1] = i2
        gts_ref[2 * b] = 1.0 / (1.0 + d)
        gts_ref[2 * b + 1] = d / (1.0 + d)


def _expert_body(idx_ref, gts_ref, x_ref, w0_ref, w1_ref, b0_ref, b1_ref, o_ref):
    b = pl.program_id(0)
    xb = x_ref[0]
    w = jnp.concatenate([w0_ref[0], w1_ref[0]], axis=1)
    a = jnp.dot(xb.astype(jnp.bfloat16), w.astype(jnp.bfloat16),
                preferred_element_type=jnp.float32)
    g0 = gts_ref[2 * b]
    g1 = gts_ref[2 * b + 1]
    a0 = a[:, :_D] + (b0_ref[0] + jnp.log(g0))
    a1 = a[:, _D:] + (b1_ref[0] + jnp.log(g1))
    o_ref[0] = jnp.logaddexp(a0, a1)


def _full(shape):
    return pl.BlockSpec(shape, lambda i: (0,) * len(shape))


def kernel(x, params):
    h0 = _morton(x.reshape(_B * _T, _HH, _WW, _D))
    p1 = h0.reshape(_B * _T * _HH * _WW // 4, 4 * _D)

    wms, b2s, g2s, t2s = [], [], [], []
    for i in range(5):
        cw = params["conv_w"][i]
        wms.append(cw.transpose(2, 3, 1, 0).reshape(-1, cw.shape[0]))
        b2s.append(params["conv_b"][i].reshape(1, -1))
        g2s.append(params["ln_g"][i].reshape(1, -1))
        t2s.append(params["ln_b"][i].reshape(1, -1))

    in_specs_a = [pl.BlockSpec((8192, 256), lambda i: (i, 0))]
    for arrs in (wms[:4], b2s[:4], g2s[:4], t2s[:4]):
        for a in arrs:
            in_specs_a.append(_full(a.shape))
    h4 = pl.pallas_call(
        _tower_body,
        grid=(3,),
        in_specs=in_specs_a,
        out_specs=pl.BlockSpec((128, 1024), lambda i: (i, 0)),
        out_shape=jax.ShapeDtypeStruct((_B * _T * 4, 1024), jnp.float32),
    )(p1, *wms[:4], *b2s[:4], *g2s[:4], *t2s[:4])

    head_in = [h4, wms[4], b2s[4], g2s[4], t2s[4], params["fuse_w"],
               params["fuse_b"].reshape(1, -1), _CBIG, _SBIG, params["w_gate"]]
    idx8, gts8 = pl.pallas_call(
        _head_body,
        out_specs=(pl.BlockSpec(memory_space=pltpu.SMEM),
                   pl.BlockSpec(memory_space=pltpu.SMEM)),
        out_shape=(jax.ShapeDtypeStruct((2 * _B,), jnp.int32),
                   jax.ShapeDtypeStruct((2 * _B,), jnp.float32)),
    )(*head_in)
    x3 = x.reshape(_B, _TOK, _D)
    eb3 = params["expert_b"].reshape(_NE, 1, _D)
    tt = 12288
    grid_spec = pltpu.PrefetchScalarGridSpec(
        num_scalar_prefetch=2,
        grid=(_B, _TOK // tt),
        in_specs=[
            pl.BlockSpec((1, tt, _D), lambda b, t, idx, gts: (b, t, 0)),
            pl.BlockSpec((1, _D, _D), lambda b, t, idx, gts: (idx[2 * b], 0, 0)),
            pl.BlockSpec((1, _D, _D), lambda b, t, idx, gts: (idx[2 * b + 1], 0, 0)),
            pl.BlockSpec((1, 1, _D), lambda b, t, idx, gts: (idx[2 * b], 0, 0)),
            pl.BlockSpec((1, 1, _D), lambda b, t, idx, gts: (idx[2 * b + 1], 0, 0)),
        ],
        out_specs=pl.BlockSpec((1, tt, _D), lambda b, t, idx, gts: (b, t, 0)),
    )
    out = pl.pallas_call(
        _expert_body,
        grid_spec=grid_spec,
        out_shape=jax.ShapeDtypeStruct((_B, _TOK, _D), jnp.float32),
    )(idx8, gts8, x3, params["expert_w"], params["expert_w"], eb3, eb3)
    return out.reshape(_B, _T, _HH, _WW, _D)


# speedup vs baseline: 1.0660x; 1.0008x over previous
import numpy as np
import jax
import jax.numpy as jnp
from jax.experimental import pallas as pl
from jax.experimental.pallas import tpu as pltpu

_B = 4
_T = 24
_HH = 32
_WW = 32
_D = 64
_NE = 7
_FPAD = 16
_TOK = _T * _HH * _WW

_t = np.arange(_T)
_f = np.arange(1, _T // 2 + 1)
_ang = 2.0 * np.pi * _f[:, None] * _t[None, :] / _T
_Cp = np.zeros((_FPAD, _T), np.float32)
_Sp = np.zeros((_FPAD, _T), np.float32)
_Cp[: _T // 2] = (np.cos(_ang) / np.sqrt(_T)).astype(np.float32)
_Sp[: _T // 2] = (np.sin(_ang) / np.sqrt(_T)).astype(np.float32)
_CBIG = np.zeros((_B * _FPAD, _B * _T), np.float32)
_SBIG = np.zeros((_B * _FPAD, _B * _T), np.float32)
for _b in range(_B):
    _CBIG[_b * _FPAD:(_b + 1) * _FPAD, _b * _T:(_b + 1) * _T] = _Cp
    _SBIG[_b * _FPAD:(_b + 1) * _FPAD, _b * _T:(_b + 1) * _T] = _Sp


def _morton(x):
    n, hh, ww, c = x.shape
    x = x.reshape(n, 2, 2, 2, 2, 2, 2, 2, 2, 2, 2, c)
    x = x.transpose(0, 1, 6, 2, 7, 3, 8, 4, 9, 5, 10, 11)
    return x.reshape(n * hh * ww, c)


def _ln_gelu(h, g, beta):
    mu = jnp.mean(h, axis=1, keepdims=True)
    var = jnp.mean((h - mu) ** 2, axis=1, keepdims=True)
    hn = (h - mu) * jax.lax.rsqrt(var + 1e-5)
    return jax.nn.gelu(hn * g + beta)


_ROWS_A = (12288, 3072, 768, 192)


def _tower_body(p_ref, w1, w2, w3, w4, b1, b2, b3, b4,
                g1, g2, g3, g4, t1, t2, t3, t4, o_ref):
    v = p_ref[...]
    for li, (w, b, g, t) in enumerate(
            ((w1, b1, g1, t1), (w2, b2, g2, t2),
             (w3, b3, g3, t3), (w4, b4, g4, t4))):
        if li > 0:
            v = v.reshape(_ROWS_A[li], v.shape[1] * 4)
        h = jnp.dot(v, w[...], preferred_element_type=jnp.float32) + b[...]
        v = _ln_gelu(h, g[...], t[...])
    o_ref[...] = v


def _head_body(p_ref, w5, b5, g5, t5, fw_ref, fb_ref,
               cb_ref, sb_ref, wg_ref, idx_ref, gts_ref):
    v = p_ref[...].reshape(_B * _T, 4096)
    h5 = _ln_gelu(jnp.dot(v, w5[...], preferred_element_type=jnp.float32)
                  + b5[...], g5[...], t5[...])
    fused = jax.lax.dot_general(
        h5, fw_ref[...], (((1,), (1,)), ((), ())),
        preferred_element_type=jnp.float32) + fb_ref[...]
    re = jnp.dot(cb_ref[...], fused, preferred_element_type=jnp.float32)
    im = jnp.dot(sb_ref[...], fused, preferred_element_type=jnp.float32)
    amp = jnp.mean(jnp.sqrt(re * re + im * im), axis=1, keepdims=True)
    ii = jax.lax.broadcasted_iota(jnp.int32, (1, _NE), 1)
    for b in range(_B):
        a_b = amp[_FPAD * b:_FPAD * b + _T // 2]
        lg = jnp.sum(a_b * wg_ref[...], axis=0, keepdims=True)
        m1 = jnp.max(lg)
        i1 = jnp.min(jnp.where(lg == m1, ii, _NE))
        lg2 = jnp.where(ii == i1, jnp.float32(-1e30), lg)
        m2 = jnp.max(lg2)
        i2 = jnp.min(jnp.where(lg2 == m2, ii, _NE))
        d = jnp.exp(m2 - m1)
        idx_ref[2 * b] = i1
        idx_ref[2 * b + 1] = i2
        gts_ref[2 * b] = 1.0 / (1.0 + d)
        gts_ref[2 * b + 1] = d / (1.0 + d)


def _expert_body(idx_ref, gts_ref, x_ref, w0_ref, w1_ref, b0_ref, b1_ref, o_ref):
    b = pl.program_id(0)
    xb = x_ref[0]
    w = jnp.concatenate([w0_ref[0], w1_ref[0]], axis=1)
    a = jnp.dot(xb.astype(jnp.bfloat16), w.astype(jnp.bfloat16),
                preferred_element_type=jnp.float32)
    g0 = gts_ref[2 * b]
    g1 = gts_ref[2 * b + 1]
    a0 = a[:, :_D] + (b0_ref[0] + jnp.log(g0))
    a1 = a[:, _D:] + (b1_ref[0] + jnp.log(g1))
    o_ref[0] = jnp.logaddexp(a0, a1)


def _full(shape):
    return pl.BlockSpec(shape, lambda i: (0,) * len(shape))


def kernel(x, params):
    h0 = _morton(x.reshape(_B * _T, _HH, _WW, _D))
    p1 = h0.reshape(_B * _T * _HH * _WW // 4, 4 * _D)

    wms, b2s, g2s, t2s = [], [], [], []
    for i in range(5):
        cw = params["conv_w"][i]
        wms.append(cw.transpose(2, 3, 1, 0).reshape(-1, cw.shape[0]))
        b2s.append(params["conv_b"][i].reshape(1, -1))
        g2s.append(params["ln_g"][i].reshape(1, -1))
        t2s.append(params["ln_b"][i].reshape(1, -1))

    in_specs_a = [pl.BlockSpec((12288, 256), lambda i: (i, 0))]
    for arrs in (wms[:4], b2s[:4], g2s[:4], t2s[:4]):
        for a in arrs:
            in_specs_a.append(_full(a.shape))
    h4 = pl.pallas_call(
        _tower_body,
        grid=(2,),
        in_specs=in_specs_a,
        out_specs=pl.BlockSpec((192, 1024), lambda i: (i, 0)),
        out_shape=jax.ShapeDtypeStruct((_B * _T * 4, 1024), jnp.float32),
    )(p1, *wms[:4], *b2s[:4], *g2s[:4], *t2s[:4])

    head_in = [h4, wms[4], b2s[4], g2s[4], t2s[4], params["fuse_w"],
               params["fuse_b"].reshape(1, -1), _CBIG, _SBIG, params["w_gate"]]
    idx8, gts8 = pl.pallas_call(
        _head_body,
        out_specs=(pl.BlockSpec(memory_space=pltpu.SMEM),
                   pl.BlockSpec(memory_space=pltpu.SMEM)),
        out_shape=(jax.ShapeDtypeStruct((2 * _B,), jnp.int32),
                   jax.ShapeDtypeStruct((2 * _B,), jnp.float32)),
    )(*head_in)
    x3 = x.reshape(_B, _TOK, _D)
    eb3 = params["expert_b"].reshape(_NE, 1, _D)
    tt = 12288
    grid_spec = pltpu.PrefetchScalarGridSpec(
        num_scalar_prefetch=2,
        grid=(_B, _TOK // tt),
        in_specs=[
            pl.BlockSpec((1, tt, _D), lambda b, t, idx, gts: (b, t, 0)),
            pl.BlockSpec((1, _D, _D), lambda b, t, idx, gts: (idx[2 * b], 0, 0)),
            pl.BlockSpec((1, _D, _D), lambda b, t, idx, gts: (idx[2 * b + 1], 0, 0)),
            pl.BlockSpec((1, 1, _D), lambda b, t, idx, gts: (idx[2 * b], 0, 0)),
            pl.BlockSpec((1, 1, _D), lambda b, t, idx, gts: (idx[2 * b + 1], 0, 0)),
        ],
        out_specs=pl.BlockSpec((1, tt, _D), lambda b, t, idx, gts: (b, t, 0)),
    )
    out = pl.pallas_call(
        _expert_body,
        grid_spec=grid_spec,
        out_shape=jax.ShapeDtypeStruct((_B, _TOK, _D), jnp.float32),
    )(idx8, gts8, x3, params["expert_w"], params["expert_w"], eb3, eb3)
    return out.reshape(_B, _T, _HH, _WW, _D)
